# Initial kernel scaffold; baseline (speedup 1.0000x reference)
import jax
import jax.numpy as jnp
from jax.experimental import pallas as pl


def kernel(x, edge_list, edge_attr, W1, b1, W_fc, b_fc):
    raise NotImplementedError("write your pallas kernel here")


# speedup vs baseline: 106.2557x; 106.2557x over previous
import functools

import jax
import jax.numpy as jnp
from jax import lax
from jax.experimental import pallas as pl
from jax.experimental.pallas import tpu as pltpu
from jax.experimental.pallas import tpu_sc as plsc

_N = 10000
_E = 320000
_NC = 2
_NS = 16
_NW = _NC * _NS
_EPT = _E // _NW
_L = 16


def _sc_mesh():
    return plsc.VectorSubcoreMesh(
        core_axis_name="c", subcore_axis_name="s",
        num_cores=_NC, num_subcores=_NS)


def _worker_id():
    return lax.axis_index("s") * _NC + lax.axis_index("c")


def _zero_vmem(ref, n):
    zv = jnp.zeros((_L,), jnp.float32)

    def body(i, carry):
        ref[pl.ds(i * _L, _L)] = zv
        return carry

    lax.fori_loop(0, n // _L, body, 0)


def _deg_body(col_hbm, w_hbm, out_hbm, col_v, w_v, acc_v):
    wid = _worker_id()
    base = wid * _EPT
    pltpu.sync_copy(col_hbm.at[pl.ds(base, _EPT)], col_v)
    pltpu.sync_copy(w_hbm.at[pl.ds(base, _EPT)], w_v)
    _zero_vmem(acc_v, _N)

    def body(i, carry):
        o = i * _L
        c = col_v[pl.ds(o, _L)]
        ww = w_v[pl.ds(o, _L)]
        plsc.addupdate_scatter(acc_v, [c], ww)
        return carry

    lax.fori_loop(0, _EPT // _L, body, 0)
    pltpu.sync_copy(acc_v, out_hbm.at[wid])


_deg_call = functools.partial(
    pl.kernel,
    out_type=jax.ShapeDtypeStruct((_NW, _N), jnp.float32),
    mesh=_sc_mesh(),
    compiler_params=pltpu.CompilerParams(needs_layout_passes=False),
    scratch_types=[
        pltpu.VMEM((_EPT,), jnp.int32),
        pltpu.VMEM((_EPT,), jnp.float32),
        pltpu.VMEM((_N,), jnp.float32),
    ],
)(_deg_body)


def _msg_body(row_hbm, col_hbm, w_hbm, g2_hbm, a0_hbm, a1_hbm,
              row_v, col_v, w_v, g0_v, g1_v, a0_v, a1_v):
    wid = _worker_id()
    base = wid * _EPT
    pltpu.sync_copy(row_hbm.at[pl.ds(base, _EPT)], row_v)
    pltpu.sync_copy(col_hbm.at[pl.ds(base, _EPT)], col_v)
    pltpu.sync_copy(w_hbm.at[pl.ds(base, _EPT)], w_v)
    pltpu.sync_copy(g2_hbm.at[0], g0_v)
    pltpu.sync_copy(g2_hbm.at[1], g1_v)
    _zero_vmem(a0_v, _N)
    _zero_vmem(a1_v, _N)

    def body(i, carry):
        o = i * _L
        r = row_v[pl.ds(o, _L)]
        c = col_v[pl.ds(o, _L)]
        ww = w_v[pl.ds(o, _L)]
        m0 = ww * plsc.load_gather(g0_v, [r])
        m1 = ww * plsc.load_gather(g1_v, [r])
        plsc.addupdate_scatter(a0_v, [c], m0)
        plsc.addupdate_scatter(a1_v, [c], m1)
        return carry

    lax.fori_loop(0, _EPT // _L, body, 0)
    pltpu.sync_copy(a0_v, a0_hbm.at[wid])
    pltpu.sync_copy(a1_v, a1_hbm.at[wid])


_msg_call = functools.partial(
    pl.kernel,
    out_type=(
        jax.ShapeDtypeStruct((_NW, _N), jnp.float32),
        jax.ShapeDtypeStruct((_NW, _N), jnp.float32),
    ),
    mesh=_sc_mesh(),
    compiler_params=pltpu.CompilerParams(needs_layout_passes=False),
    scratch_types=[
        pltpu.VMEM((_EPT,), jnp.int32),
        pltpu.VMEM((_EPT,), jnp.int32),
        pltpu.VMEM((_EPT,), jnp.float32),
        pltpu.VMEM((_N,), jnp.float32),
        pltpu.VMEM((_N,), jnp.float32),
        pltpu.VMEM((_N,), jnp.float32),
        pltpu.VMEM((_N,), jnp.float32),
    ],
)(_msg_body)


def _mm_body(x_ref, w1_ref, h_ref):
    h_ref[...] = jnp.dot(x_ref[...], w1_ref[...],
                         preferred_element_type=jnp.float32)


def _mm_call(x, w1):
    return pl.pallas_call(
        _mm_body,
        out_shape=jax.ShapeDtypeStruct((_N, 2), jnp.float32),
    )(x, w1)


def _prep_body(degp_ref, h2_ref, dis_ref, g2_ref):
    deg = jnp.sum(degp_ref[...], axis=0, keepdims=True) + 1.0
    dis = jnp.where(deg > 0, lax.rsqrt(jnp.maximum(deg, 1e-12)), 0.0)
    dis_ref[...] = dis
    g2_ref[...] = dis * h2_ref[...]


def _prep_call(degp, h2):
    return pl.pallas_call(
        _prep_body,
        out_shape=(
            jax.ShapeDtypeStruct((1, _N), jnp.float32),
            jax.ShapeDtypeStruct((2, _N), jnp.float32),
        ),
    )(degp, h2)


def _final_body(a0p_ref, a1p_ref, g2_ref, dis_ref, wfc_ref, b1_ref, bfc_ref,
                y_ref):
    a0 = jnp.sum(a0p_ref[...], axis=0, keepdims=True)
    a1 = jnp.sum(a1p_ref[...], axis=0, keepdims=True)
    a = jnp.concatenate([a0, a1], axis=0)
    out = dis_ref[...] * (a + g2_ref[...]) + b1_ref[...]
    out = jnp.maximum(out, 0.0)
    s = jnp.sum(out * wfc_ref[...], keepdims=True).reshape(1, 1)
    y_ref[...] = jax.nn.sigmoid(s + bfc_ref[...])


def _final_call(a0p, a1p, g2, dis, wfc2, b1, bfc):
    return pl.pallas_call(
        _final_body,
        out_shape=jax.ShapeDtypeStruct((1, 1), jnp.float32),
    )(a0p, a1p, g2, dis, wfc2, b1, bfc)


def kernel(x, edge_list, edge_attr, W1, b1, W_fc, b_fc):
    row = edge_list[0]
    col = edge_list[1]
    degp = _deg_call(col, edge_attr)
    h = _mm_call(x, W1)
    h2 = h.T
    dis, g2 = _prep_call(degp, h2)
    a0p, a1p = _msg_call(row, col, edge_attr, g2)
    wfc2 = W_fc.reshape(_N, 2).T
    y = _final_call(a0p, a1p, g2, dis, wfc2,
                    b1.reshape(2, 1), b_fc.reshape(1, 1))
    return y[0, 0]


# speedup vs baseline: 136.1547x; 1.2814x over previous
import functools

import jax
import jax.numpy as jnp
from jax import lax
from jax.experimental import pallas as pl
from jax.experimental.pallas import tpu as pltpu
from jax.experimental.pallas import tpu_sc as plsc

_N = 10000
_E = 320000
_NC = 2
_NS = 16
_NW = _NC * _NS
_EPT = _E // _NW
_L = 16
_N2 = 2 * _N
_UNROLL = 8


def _sc_mesh():
    return plsc.VectorSubcoreMesh(
        core_axis_name="c", subcore_axis_name="s",
        num_cores=_NC, num_subcores=_NS)


def _worker_id():
    return lax.axis_index("s") * _NC + lax.axis_index("c")


def _zero_vmem(ref, n):
    zv = jnp.zeros((_L,), jnp.float32)

    @plsc.parallel_loop(0, n // _L, unroll=_UNROLL)
    def _(i):
        ref[pl.ds(i * _L, _L)] = zv


def _deg_body(el_hbm, w_hbm, out_hbm, col_v, w_v, acc_v):
    wid = _worker_id()
    base = wid * _EPT
    pltpu.sync_copy(el_hbm.at[pl.ds(_E + base, _EPT)], col_v)
    pltpu.sync_copy(w_hbm.at[pl.ds(base, _EPT)], w_v)
    _zero_vmem(acc_v, _N2)

    @plsc.parallel_loop(0, _EPT // _L, unroll=_UNROLL)
    def _(i):
        o = i * _L
        c2 = col_v[pl.ds(o, _L)] * 2
        ww = w_v[pl.ds(o, _L)]
        plsc.addupdate_scatter(acc_v, [c2], ww)
        plsc.addupdate_scatter(acc_v, [c2 + 1], ww)

    pltpu.sync_copy(acc_v, out_hbm.at[wid])


_deg_call = functools.partial(
    pl.kernel,
    out_type=jax.ShapeDtypeStruct((_NW, _N2), jnp.float32),
    mesh=_sc_mesh(),
    compiler_params=pltpu.CompilerParams(needs_layout_passes=False),
    scratch_types=[
        pltpu.VMEM((_EPT,), jnp.int32),
        pltpu.VMEM((_EPT,), jnp.float32),
        pltpu.VMEM((_N2,), jnp.float32),
    ],
)(_deg_body)


def _msg_body(el_hbm, w_hbm, g_hbm, out_hbm,
              row_v, col_v, w_v, g_v, acc_v):
    wid = _worker_id()
    base = wid * _EPT
    pltpu.sync_copy(el_hbm.at[pl.ds(base, _EPT)], row_v)
    pltpu.sync_copy(el_hbm.at[pl.ds(_E + base, _EPT)], col_v)
    pltpu.sync_copy(w_hbm.at[pl.ds(base, _EPT)], w_v)
    pltpu.sync_copy(g_hbm.at[0], g_v)
    _zero_vmem(acc_v, _N2)

    @plsc.parallel_loop(0, _EPT // _L, unroll=_UNROLL)
    def _(i):
        o = i * _L
        r2 = row_v[pl.ds(o, _L)] * 2
        c2 = col_v[pl.ds(o, _L)] * 2
        ww = w_v[pl.ds(o, _L)]
        m0 = ww * plsc.load_gather(g_v, [r2])
        m1 = ww * plsc.load_gather(g_v, [r2 + 1])
        plsc.addupdate_scatter(acc_v, [c2], m0)
        plsc.addupdate_scatter(acc_v, [c2 + 1], m1)

    pltpu.sync_copy(acc_v, out_hbm.at[wid])


_msg_call = functools.partial(
    pl.kernel,
    out_type=jax.ShapeDtypeStruct((_NW, _N2), jnp.float32),
    mesh=_sc_mesh(),
    compiler_params=pltpu.CompilerParams(needs_layout_passes=False),
    scratch_types=[
        pltpu.VMEM((_EPT,), jnp.int32),
        pltpu.VMEM((_EPT,), jnp.int32),
        pltpu.VMEM((_EPT,), jnp.float32),
        pltpu.VMEM((_N2,), jnp.float32),
        pltpu.VMEM((_N2,), jnp.float32),
    ],
)(_msg_body)


def _mm_body(x_ref, w1_ref, h_ref):
    h_ref[...] = jnp.dot(x_ref[...], w1_ref[...],
                         preferred_element_type=jnp.float32)


def _mm_call(x, w1):
    return pl.pallas_call(
        _mm_body,
        out_shape=jax.ShapeDtypeStruct((_N, 2), jnp.float32),
    )(x, w1)


def _prep_body(degp_ref, hf_ref, dis_ref, g_ref):
    deg = jnp.sum(degp_ref[...], axis=0, keepdims=True) + 1.0
    dis = jnp.where(deg > 0, lax.rsqrt(jnp.maximum(deg, 1e-12)), 0.0)
    dis_ref[...] = dis
    g_ref[...] = dis * hf_ref[...]


def _prep_call(degp, hf):
    return pl.pallas_call(
        _prep_body,
        out_shape=(
            jax.ShapeDtypeStruct((1, _N2), jnp.float32),
            jax.ShapeDtypeStruct((1, _N2), jnp.float32),
        ),
    )(degp, hf)


def _final_body(ap_ref, g_ref, dis_ref, wfc_ref, b1f_ref, bfc_ref, y_ref):
    a = jnp.sum(ap_ref[...], axis=0, keepdims=True)
    out = dis_ref[...] * (a + g_ref[...]) + b1f_ref[...]
    out = jnp.maximum(out, 0.0)
    s = jnp.sum(out * wfc_ref[...], keepdims=True).reshape(1, 1)
    y_ref[...] = jax.nn.sigmoid(s + bfc_ref[...])


def _final_call(ap, g, dis, wfc, b1f, bfc):
    return pl.pallas_call(
        _final_body,
        out_shape=jax.ShapeDtypeStruct((1, 1), jnp.float32),
    )(ap, g, dis, wfc, b1f, bfc)


def kernel(x, edge_list, edge_attr, W1, b1, W_fc, b_fc):
    el_flat = edge_list.reshape(2 * _E)
    degp = _deg_call(el_flat, edge_attr)
    h = _mm_call(x, W1)
    hf = h.reshape(1, _N2)
    dis2, gf = _prep_call(degp, hf)
    ap = _msg_call(el_flat, edge_attr, gf)
    b1f = jnp.tile(b1, _N).reshape(1, _N2)
    y = _final_call(ap, gf, dis2, W_fc, b1f, b_fc.reshape(1, 1))
    return y[0, 0]


# speedup vs baseline: 161.3620x; 1.1851x over previous
import functools

import jax
import jax.numpy as jnp
from jax import lax
from jax.experimental import pallas as pl
from jax.experimental.pallas import tpu as pltpu
from jax.experimental.pallas import tpu_sc as plsc

_N = 10000
_E = 320000
_NC = 2
_NS = 16
_NW = _NC * _NS
_ECH = 9984
_EBUF = _E - (_NW - 1) * _ECH
_L = 16
_N2 = 2 * _N
_NPT = 384
_NP = _NW * _NPT
_UNROLL = 8


def _sc_mesh():
    return plsc.VectorSubcoreMesh(
        core_axis_name="c", subcore_axis_name="s",
        num_cores=_NC, num_subcores=_NS)


def _worker_id():
    return lax.axis_index("s") * _NC + lax.axis_index("c")


def _zero_vmem(ref, n):
    zv = jnp.zeros((_L,), jnp.float32)

    @plsc.parallel_loop(0, n // _L, unroll=_UNROLL)
    def _(i):
        ref[pl.ds(i * _L, _L)] = zv


def _deg_body(el_hbm, w_hbm, out_hbm, rc_v, w_v, acc_v):
    wid = _worker_id()
    base = wid * _ECH
    nown = jnp.where(wid == _NW - 1, _EBUF, _ECH)
    pltpu.sync_copy(el_hbm.at[:, pl.ds(base, _EBUF)], rc_v)
    pltpu.sync_copy(w_hbm.at[pl.ds(base, _EBUF)], w_v)
    _zero_vmem(acc_v, _N)

    iota = lax.iota(jnp.int32, _L)

    @plsc.parallel_loop(0, _EBUF // _L, unroll=_UNROLL)
    def _(i):
        o = i * _L
        mask = (o + iota) < nown
        c = rc_v[1, pl.ds(o, _L)]
        ww = w_v[pl.ds(o, _L)]
        plsc.addupdate_scatter(acc_v, [c], ww, mask=mask)

    pltpu.sync_copy(acc_v, out_hbm.at[wid])


_deg_call = functools.partial(
    pl.kernel,
    out_type=jax.ShapeDtypeStruct((_NW, _N), jnp.float32),
    mesh=_sc_mesh(),
    compiler_params=pltpu.CompilerParams(needs_layout_passes=False),
    scratch_types=[
        pltpu.VMEM((2, _EBUF), jnp.int32),
        pltpu.VMEM((_EBUF,), jnp.float32),
        pltpu.VMEM((_N,), jnp.float32),
    ],
)(_deg_body)


def _msg_body(el_hbm, w_hbm, g2_hbm, dis_hbm, gb_hbm, out_hbm,
              rc_v, w_v, g_v, dis_v, gb_v, acc_v):
    wid = _worker_id()
    base = wid * _ECH
    nown = jnp.where(wid == _NW - 1, _EBUF, _ECH)
    nb = wid * _NPT
    pltpu.sync_copy(el_hbm.at[:, pl.ds(base, _EBUF)], rc_v)
    pltpu.sync_copy(w_hbm.at[pl.ds(base, _EBUF)], w_v)
    pltpu.sync_copy(g2_hbm, g_v)
    pltpu.sync_copy(dis_hbm.at[0], dis_v)
    pltpu.sync_copy(gb_hbm.at[:, pl.ds(nb, _NPT)], gb_v)
    _zero_vmem(acc_v, _N2)

    iota = lax.iota(jnp.int32, _L)
    zeros16 = jnp.zeros((_L,), jnp.int32)
    ones16 = jnp.ones((_L,), jnp.int32)

    @plsc.parallel_loop(0, _NPT // _L, unroll=4)
    def _(j):
        jj = j * _L
        n16 = nb + jj + iota
        mask = n16 < _N
        n2 = n16 * 2
        plsc.store_scatter(acc_v, [n2], gb_v[0, pl.ds(jj, _L)], mask=mask)
        plsc.store_scatter(acc_v, [n2 + 1], gb_v[1, pl.ds(jj, _L)], mask=mask)

    @plsc.parallel_loop(0, _EBUF // _L, unroll=_UNROLL)
    def _(i):
        o = i * _L
        mask = (o + iota) < nown
        r = rc_v[0, pl.ds(o, _L)]
        c = rc_v[1, pl.ds(o, _L)]
        ww = w_v[pl.ds(o, _L)]
        wd = ww * plsc.load_gather(dis_v, [c])
        m0 = wd * plsc.load_gather(g_v, [zeros16, r])
        m1 = wd * plsc.load_gather(g_v, [ones16, r])
        c2 = c * 2
        plsc.addupdate_scatter(acc_v, [c2], m0, mask=mask)
        plsc.addupdate_scatter(acc_v, [c2 + 1], m1, mask=mask)

    pltpu.sync_copy(acc_v, out_hbm.at[wid])


_msg_call = functools.partial(
    pl.kernel,
    out_type=jax.ShapeDtypeStruct((_NW, _N2), jnp.float32),
    mesh=_sc_mesh(),
    compiler_params=pltpu.CompilerParams(needs_layout_passes=False),
    scratch_types=[
        pltpu.VMEM((2, _EBUF), jnp.int32),
        pltpu.VMEM((_EBUF,), jnp.float32),
        pltpu.VMEM((2, _N), jnp.float32),
        pltpu.VMEM((_N,), jnp.float32),
        pltpu.VMEM((2, _NPT), jnp.float32),
        pltpu.VMEM((_N2,), jnp.float32),
    ],
)(_msg_body)


def _mm_body(x_ref, w1_ref, h_ref):
    h_ref[...] = lax.dot_general(
        w1_ref[...], x_ref[...], (((0,), (1,)), ((), ())),
        preferred_element_type=jnp.float32)


def _mm_call(x, w1):
    return pl.pallas_call(
        _mm_body,
        out_shape=jax.ShapeDtypeStruct((2, _N), jnp.float32),
    )(x, w1)


def _prep_body(degp_ref, h2_ref, b1_ref, dis_ref, g2_ref, gb_ref):
    deg = jnp.sum(degp_ref[...], axis=0, keepdims=True) + 1.0
    dis = jnp.where(deg > 0, lax.rsqrt(jnp.maximum(deg, 1e-12)), 0.0)
    dis_ref[...] = dis
    g2 = dis * h2_ref[...]
    g2_ref[...] = g2
    gb_ref[...] = jnp.pad(dis * g2 + b1_ref[...], ((0, 0), (0, _NP - _N)))


def _prep_call(degp, h2, b1r):
    return pl.pallas_call(
        _prep_body,
        out_shape=(
            jax.ShapeDtypeStruct((1, _N), jnp.float32),
            jax.ShapeDtypeStruct((2, _N), jnp.float32),
            jax.ShapeDtypeStruct((2, _NP), jnp.float32),
        ),
    )(degp, h2, b1r)


def _final_body(ap_ref, wfc_ref, bfc_ref, y_ref):
    out = jnp.maximum(jnp.sum(ap_ref[...], axis=0, keepdims=True), 0.0)
    s = jnp.sum(out * wfc_ref[...], keepdims=True).reshape(1, 1)
    y_ref[...] = jax.nn.sigmoid(s + bfc_ref[...])


def _final_call(ap, wfc, bfc):
    return pl.pallas_call(
        _final_body,
        out_shape=jax.ShapeDtypeStruct((1, 1), jnp.float32),
    )(ap, wfc, bfc)


def kernel(x, edge_list, edge_attr, W1, b1, W_fc, b_fc):
    degp = _deg_call(edge_list, edge_attr)
    h2 = _mm_call(x, W1)
    dis, g2, gb2 = _prep_call(degp, h2, b1.reshape(2, 1))
    ap = _msg_call(edge_list, edge_attr, g2, dis, gb2)
    y = _final_call(ap, W_fc, b_fc.reshape(1, 1))
    return y[0, 0]


# speedup vs baseline: 174.1292x; 1.0791x over previous
import functools

import jax
import jax.numpy as jnp
from jax import lax
from jax.experimental import pallas as pl
from jax.experimental.pallas import tpu as pltpu
from jax.experimental.pallas import tpu_sc as plsc

_N = 10000
_E = 320000
_NC = 2
_NS = 16
_NW = _NC * _NS
_ECH = 9984
_EBUF = _E - (_NW - 1) * _ECH
_L = 16
_N2 = 2 * _N
_NPT = 384
_NP = _NW * _NPT
_UNROLL = 8


def _sc_mesh():
    return plsc.VectorSubcoreMesh(
        core_axis_name="c", subcore_axis_name="s",
        num_cores=_NC, num_subcores=_NS)


def _worker_id():
    return lax.axis_index("s") * _NC + lax.axis_index("c")


def _zero_vmem(ref, n):
    zv = jnp.zeros((_L,), jnp.float32)

    @plsc.parallel_loop(0, n // _L, unroll=_UNROLL)
    def _(i):
        ref[pl.ds(i * _L, _L)] = zv


def _deg_body(el_hbm, w_hbm, out_hbm, rc_v, w_v, acc_v, sem):
    wid = _worker_id()
    base = wid * _ECH
    nown = jnp.where(wid == _NW - 1, _EBUF, _ECH)
    cp1 = pltpu.async_copy(el_hbm.at[:, pl.ds(base, _EBUF)], rc_v, sem)
    cp2 = pltpu.async_copy(w_hbm.at[pl.ds(base, _EBUF)], w_v, sem)
    _zero_vmem(acc_v, _N)
    cp1.wait()
    cp2.wait()

    iota = lax.iota(jnp.int32, _L)

    @plsc.parallel_loop(0, _EBUF // _L, unroll=_UNROLL)
    def _(i):
        o = i * _L
        mask = (o + iota) < nown
        c = rc_v[1, pl.ds(o, _L)]
        ww = w_v[pl.ds(o, _L)]
        plsc.addupdate_scatter(acc_v, [c], ww, mask=mask)

    pltpu.sync_copy(acc_v, out_hbm.at[wid])


_deg_call = functools.partial(
    pl.kernel,
    out_type=jax.ShapeDtypeStruct((_NW, _N), jnp.float32),
    mesh=_sc_mesh(),
    compiler_params=pltpu.CompilerParams(needs_layout_passes=False),
    scratch_types=[
        pltpu.VMEM((2, _EBUF), jnp.int32),
        pltpu.VMEM((_EBUF,), jnp.float32),
        pltpu.VMEM((_N,), jnp.float32),
        pltpu.SemaphoreType.DMA,
    ],
)(_deg_body)


def _msg_body(el_hbm, w_hbm, g0_hbm, g1_hbm, dis_hbm, gb_hbm, out_hbm,
              rc_v, w_v, g0_v, g1_v, dis_v, gb_v, acc_v, sem):
    wid = _worker_id()
    base = wid * _ECH
    nown = jnp.where(wid == _NW - 1, _EBUF, _ECH)
    nb = wid * _NPT
    cps = [
        pltpu.async_copy(el_hbm.at[:, pl.ds(base, _EBUF)], rc_v, sem),
        pltpu.async_copy(w_hbm.at[pl.ds(base, _EBUF)], w_v, sem),
        pltpu.async_copy(g0_hbm.at[0], g0_v, sem),
        pltpu.async_copy(g1_hbm.at[0], g1_v, sem),
        pltpu.async_copy(dis_hbm.at[0], dis_v, sem),
        pltpu.async_copy(gb_hbm.at[:, pl.ds(nb, _NPT)], gb_v, sem),
    ]
    _zero_vmem(acc_v, _N2)
    for cp in cps:
        cp.wait()

    iota = lax.iota(jnp.int32, _L)

    @plsc.parallel_loop(0, _NPT // _L, unroll=4)
    def _(j):
        jj = j * _L
        n16 = nb + jj + iota
        mask = n16 < _N
        n2 = n16 * 2
        plsc.store_scatter(acc_v, [n2], gb_v[0, pl.ds(jj, _L)], mask=mask)
        plsc.store_scatter(acc_v, [n2 + 1], gb_v[1, pl.ds(jj, _L)], mask=mask)

    @plsc.parallel_loop(0, _EBUF // _L, unroll=_UNROLL)
    def _(i):
        o = i * _L
        mask = (o + iota) < nown
        r = rc_v[0, pl.ds(o, _L)]
        c = rc_v[1, pl.ds(o, _L)]
        ww = w_v[pl.ds(o, _L)]
        wd = ww * plsc.load_gather(dis_v, [c])
        m0 = wd * plsc.load_gather(g0_v, [r])
        m1 = wd * plsc.load_gather(g1_v, [r])
        c2 = c * 2
        plsc.addupdate_scatter(acc_v, [c2], m0, mask=mask)
        plsc.addupdate_scatter(acc_v, [c2 + 1], m1, mask=mask)

    pltpu.sync_copy(acc_v, out_hbm.at[wid])


_msg_call = functools.partial(
    pl.kernel,
    out_type=jax.ShapeDtypeStruct((_NW, _N2), jnp.float32),
    mesh=_sc_mesh(),
    compiler_params=pltpu.CompilerParams(needs_layout_passes=False),
    scratch_types=[
        pltpu.VMEM((2, _EBUF), jnp.int32),
        pltpu.VMEM((_EBUF,), jnp.float32),
        pltpu.VMEM((_N,), jnp.float32),
        pltpu.VMEM((_N,), jnp.float32),
        pltpu.VMEM((_N,), jnp.float32),
        pltpu.VMEM((2, _NPT), jnp.float32),
        pltpu.VMEM((_N2,), jnp.float32),
        pltpu.SemaphoreType.DMA,
    ],
)(_msg_body)


def _mm_body(x_ref, w1_ref, h_ref):
    h_ref[...] = lax.dot_general(
        w1_ref[...], x_ref[...], (((0,), (1,)), ((), ())),
        preferred_element_type=jnp.float32)


def _mm_call(x, w1):
    return pl.pallas_call(
        _mm_body,
        out_shape=jax.ShapeDtypeStruct((2, _N), jnp.float32),
    )(x, w1)


def _prep_body(degp_ref, h2_ref, b1_ref, dis_ref, g0_ref, g1_ref, gb_ref):
    deg = jnp.sum(degp_ref[...], axis=0, keepdims=True) + 1.0
    dis = jnp.where(deg > 0, lax.rsqrt(jnp.maximum(deg, 1e-12)), 0.0)
    dis_ref[...] = dis
    g2 = dis * h2_ref[...]
    g0_ref[...] = g2[0:1, :]
    g1_ref[...] = g2[1:2, :]
    gb_ref[...] = jnp.pad(dis * g2 + b1_ref[...], ((0, 0), (0, _NP - _N)))


def _prep_call(degp, h2, b1r):
    return pl.pallas_call(
        _prep_body,
        out_shape=(
            jax.ShapeDtypeStruct((1, _N), jnp.float32),
            jax.ShapeDtypeStruct((1, _N), jnp.float32),
            jax.ShapeDtypeStruct((1, _N), jnp.float32),
            jax.ShapeDtypeStruct((2, _NP), jnp.float32),
        ),
    )(degp, h2, b1r)


def _final_body(ap_ref, wfc_ref, bfc_ref, y_ref):
    out = jnp.maximum(jnp.sum(ap_ref[...], axis=0, keepdims=True), 0.0)
    s = jnp.sum(out * wfc_ref[...], keepdims=True).reshape(1, 1)
    y_ref[...] = jax.nn.sigmoid(s + bfc_ref[...])


def _final_call(ap, wfc, bfc):
    return pl.pallas_call(
        _final_body,
        out_shape=jax.ShapeDtypeStruct((1, 1), jnp.float32),
    )(ap, wfc, bfc)


def kernel(x, edge_list, edge_attr, W1, b1, W_fc, b_fc):
    degp = _deg_call(edge_list, edge_attr)
    h2 = _mm_call(x, W1)
    dis, g0, g1, gb2 = _prep_call(degp, h2, b1.reshape(2, 1))
    ap = _msg_call(edge_list, edge_attr, g0, g1, dis, gb2)
    y = _final_call(ap, W_fc, b_fc.reshape(1, 1))
    return y[0, 0]


# speedup vs baseline: 175.2161x; 1.0062x over previous
import functools

import jax
import jax.numpy as jnp
from jax import lax
from jax.experimental import pallas as pl
from jax.experimental.pallas import tpu as pltpu
from jax.experimental.pallas import tpu_sc as plsc

_N = 10000
_E = 320000
_NC = 2
_NS = 16
_NW = _NC * _NS
_L = 16
_N2 = 2 * _N

_DCH = 19968
_DBUF = _E - (_NS - 1) * _DCH
_MCH = _DCH // 2
_MBUF = _DBUF // 2

_NH = 10240
_STR = _NH // _NS


def _sc_mesh():
    return plsc.VectorSubcoreMesh(
        core_axis_name="c", subcore_axis_name="s",
        num_cores=_NC, num_subcores=_NS)


def _zero_vmem(ref, n):
    zv = jnp.zeros((_L,), jnp.float32)

    @plsc.parallel_loop(0, n // _L, unroll=8)
    def _(i):
        ref[pl.ds(i * _L, _L)] = zv


def _newton_rsqrt(x):
    i = plsc.bitcast(x, jnp.int32)
    y = plsc.bitcast(jnp.int32(0x5F3759DF) - (i >> 1), jnp.float32)
    for _ in range(3):
        y = y * (1.5 - 0.5 * x * y * y)
    return y


def _sc_body(el_hbm, w_hbm, h2_hbm, b1_hbm, out_hbm,
             rc_v, w_v, dd_v, tmp2_v, g0_v, g1_v, acc_v, hs_v, b1_v,
             ds_v, g0s_v, g1s_v,
             spm_deg, spm_dis, spm_g0, spm_g1, sem):
    cid = lax.axis_index("c")
    sid = lax.axis_index("s")
    wid = sid * _NC + cid

    dbase = sid * _DCH
    mlen = jnp.where(sid == _NS - 1, _MBUF, _MCH)
    nb = sid * _STR

    hbase_a = pl.multiple_of(dbase + (1 - cid) * mlen, 128)
    hbase_b = pl.multiple_of(dbase + cid * mlen, 128)

    cps = [
        pltpu.async_copy(el_hbm.at[:, pl.ds(hbase_a, _MBUF)], rc_v, sem),
        pltpu.async_copy(w_hbm.at[pl.ds(hbase_a, _MBUF)], w_v, sem),
        pltpu.async_copy(h2_hbm.at[:, pl.ds(nb, _STR)], hs_v, sem),
        pltpu.async_copy(b1_hbm.at[0], b1_v, sem),
    ]
    _zero_vmem(dd_v, _NH)
    _zero_vmem(acc_v, _N2)
    for cp in cps:
        cp.wait()

    iota = lax.iota(jnp.int32, _L)
    zeros16 = jnp.zeros((_L,), jnp.int32)
    ones16 = jnp.ones((_L,), jnp.int32)

    def _deg_loop():
        @plsc.parallel_loop(0, _MBUF // _L, unroll=8)
        def _(i):
            o = i * _L
            mask = (o + iota) < mlen
            c = rc_v[1, pl.ds(o, _L)]
            ww = w_v[pl.ds(o, _L)]
            plsc.addupdate_scatter(dd_v, [c], ww, mask=mask)

    _deg_loop()
    cp1 = pltpu.async_copy(el_hbm.at[:, pl.ds(hbase_b, _MBUF)], rc_v, sem)
    cp2 = pltpu.async_copy(w_hbm.at[pl.ds(hbase_b, _MBUF)], w_v, sem)
    cp1.wait()
    cp2.wait()
    _deg_loop()

    pltpu.sync_copy(dd_v, spm_deg.at[sid])
    plsc.subcore_barrier()
    pltpu.sync_copy(spm_deg.at[:, pl.ds(nb, _STR)], tmp2_v)

    @plsc.parallel_loop(0, _STR // _L, unroll=4)
    def _(j):
        o = j * _L
        deg = tmp2_v[0, pl.ds(o, _L)]
        for k in range(1, _NS):
            deg = deg + tmp2_v[k, pl.ds(o, _L)]
        dis = _newton_rsqrt(deg + 1.0)
        ds_v[pl.ds(o, _L)] = dis
        g0s_v[pl.ds(o, _L)] = dis * hs_v[0, pl.ds(o, _L)]
        g1s_v[pl.ds(o, _L)] = dis * hs_v[1, pl.ds(o, _L)]

    pltpu.sync_copy(ds_v, spm_dis.at[pl.ds(nb, _STR)])
    pltpu.sync_copy(g0s_v, spm_g0.at[pl.ds(nb, _STR)])
    pltpu.sync_copy(g1s_v, spm_g1.at[pl.ds(nb, _STR)])
    plsc.subcore_barrier()
    pltpu.sync_copy(spm_dis, dd_v)
    pltpu.sync_copy(spm_g0, g0_v)
    pltpu.sync_copy(spm_g1, g1_v)

    @pl.when(cid == 0)
    def _():
        b0 = plsc.load_gather(b1_v, [zeros16])
        b1b = plsc.load_gather(b1_v, [ones16])

        @plsc.parallel_loop(0, _STR // _L, unroll=4)
        def _(j):
            jj = j * _L
            n16 = nb + jj + iota
            mask = n16 < _N
            n2 = n16 * 2
            s0 = dd_v[pl.ds(nb + jj, _L)] * g0_v[pl.ds(nb + jj, _L)] + b0
            s1 = dd_v[pl.ds(nb + jj, _L)] * g1_v[pl.ds(nb + jj, _L)] + b1b
            plsc.store_scatter(acc_v, [n2], s0, mask=mask)
            plsc.store_scatter(acc_v, [n2 + 1], s1, mask=mask)

    @plsc.parallel_loop(0, _MBUF // _L, unroll=8)
    def _(i):
        o = i * _L
        mask = (o + iota) < mlen
        r = rc_v[0, pl.ds(o, _L)]
        c = rc_v[1, pl.ds(o, _L)]
        ww = w_v[pl.ds(o, _L)]
        wd = ww * plsc.load_gather(dd_v, [c])
        m0 = wd * plsc.load_gather(g0_v, [r])
        m1 = wd * plsc.load_gather(g1_v, [r])
        c2 = c * 2
        plsc.addupdate_scatter(acc_v, [c2], m0, mask=mask)
        plsc.addupdate_scatter(acc_v, [c2 + 1], m1, mask=mask)

    pltpu.sync_copy(acc_v, out_hbm.at[wid])


_sc_call = functools.partial(
    pl.kernel,
    out_type=jax.ShapeDtypeStruct((_NW, _N2), jnp.float32),
    mesh=_sc_mesh(),
    compiler_params=pltpu.CompilerParams(needs_layout_passes=False),
    scratch_types=[
        pltpu.VMEM((2, _MBUF), jnp.int32),
        pltpu.VMEM((_MBUF,), jnp.float32),
        pltpu.VMEM((_NH,), jnp.float32),
        pltpu.VMEM((_NS, _STR), jnp.float32),
        pltpu.VMEM((_NH,), jnp.float32),
        pltpu.VMEM((_NH,), jnp.float32),
        pltpu.VMEM((_N2,), jnp.float32),
        pltpu.VMEM((2, _STR), jnp.float32),
        pltpu.VMEM((_L,), jnp.float32),
        pltpu.VMEM((_STR,), jnp.float32),
        pltpu.VMEM((_STR,), jnp.float32),
        pltpu.VMEM((_STR,), jnp.float32),
        pltpu.VMEM_SHARED((_NS, _NH), jnp.float32),
        pltpu.VMEM_SHARED((_NH,), jnp.float32),
        pltpu.VMEM_SHARED((_NH,), jnp.float32),
        pltpu.VMEM_SHARED((_NH,), jnp.float32),
        pltpu.SemaphoreType.DMA,
    ],
)(_sc_body)


def _mm_body(x_ref, w1_ref, h_ref):
    h = lax.dot_general(
        w1_ref[...], x_ref[...], (((0,), (1,)), ((), ())),
        preferred_element_type=jnp.float32)
    h_ref[...] = jnp.pad(h, ((0, 0), (0, _NH - _N)))


def _mm_call(x, w1):
    return pl.pallas_call(
        _mm_body,
        out_shape=jax.ShapeDtypeStruct((2, _NH), jnp.float32),
    )(x, w1)


def _final_body(ap_ref, wfc_ref, bfc_ref, y_ref):
    out = jnp.maximum(jnp.sum(ap_ref[...], axis=0, keepdims=True), 0.0)
    s = jnp.sum(out * wfc_ref[...], keepdims=True).reshape(1, 1)
    y_ref[...] = jax.nn.sigmoid(s + bfc_ref[...])


def _final_call(ap, wfc, bfc):
    return pl.pallas_call(
        _final_body,
        out_shape=jax.ShapeDtypeStruct((1, 1), jnp.float32),
    )(ap, wfc, bfc)


def kernel(x, edge_list, edge_attr, W1, b1, W_fc, b_fc):
    h2 = _mm_call(x, W1)
    b1p = jnp.pad(b1, (0, _L - 2)).reshape(1, _L)
    ap = _sc_call(edge_list, edge_attr, h2, b1p)
    y = _final_call(ap, W_fc, b_fc.reshape(1, 1))
    return y[0, 0]
